# Initial kernel scaffold; baseline (speedup 1.0000x reference)
import jax
import jax.numpy as jnp
from jax.experimental import pallas as pl


def kernel(u, y, ind, image, U, sign_L):
    raise NotImplementedError("write your pallas kernel here")


# speedup vs baseline: 57.3328x; 57.3328x over previous
import jax
import jax.numpy as jnp
from jax.experimental import pallas as pl

_ALPHA = 0.1
_EPS = 1e-12


def _loss_kernel(u_ref, img_ref, out_ref):
    u = u_ref[...]
    img = img_ref[...]
    s = jnp.where(img > 0.0, 1.0, jnp.where(img < 0.0, -1.0, 0.0))
    ns = jnp.sqrt(jnp.sum(s * s, axis=1, keepdims=True))
    sl = s / jnp.maximum(ns, _EPS)
    ns2 = jnp.sqrt(jnp.sum(sl * sl, axis=1, keepdims=True))
    sl2 = sl / jnp.maximum(ns2, _EPS)
    nu = jnp.sqrt(jnp.sum(u * u, axis=1, keepdims=True))
    un = u / jnp.maximum(nu, _EPS)
    diff = sl2 - un
    d2 = diff * diff
    mask = jnp.where(sl * u < 0.0, 1.0, 0.0)
    per_elem = d2 + mask * d2 + _ALPHA * jnp.abs(diff)
    out_ref[...] = jnp.reshape(jnp.sum(per_elem) / u.shape[0], (1, 1))


def kernel(u, y, ind, image, U, sign_L):
    out = pl.pallas_call(
        _loss_kernel,
        out_shape=jax.ShapeDtypeStruct((1, 1), jnp.float32),
    )(u, image)
    return jnp.reshape(out, ())


# speedup vs baseline: 66.8741x; 1.1664x over previous
import jax
import jax.numpy as jnp
from jax.experimental import pallas as pl

_ALPHA = 0.1
_EPS2 = 1e-24


def _loss_kernel(u_ref, img_ref, out_ref):
    u = u_ref[...]
    img = img_ref[...]
    pos = img > 0.0
    neg = img < 0.0
    nz = jnp.where(img != 0.0, 1.0, 0.0)
    k = jnp.sum(nz, axis=1, keepdims=True)
    a = jax.lax.rsqrt(jnp.maximum(k, _EPS2))
    na = -a
    nsq = jnp.sum(u * u, axis=1, keepdims=True)
    b = jax.lax.rsqrt(jnp.maximum(nsq, _EPS2))
    sa = jnp.where(pos, a, jnp.where(neg, na, 0.0))
    diff = sa - u * b
    d2 = diff * diff
    factor = jnp.where(sa * u < 0.0, 2.0, 1.0)
    contrib = d2 * factor + _ALPHA * jnp.abs(diff)
    out_ref[...] = jnp.reshape(jnp.sum(contrib) * (1.0 / u.shape[0]), (1, 1))


def kernel(u, y, ind, image, U, sign_L):
    out = pl.pallas_call(
        _loss_kernel,
        out_shape=jax.ShapeDtypeStruct((1, 1), jnp.float32),
    )(u, image)
    return jnp.reshape(out, ())
